# initial kernel scaffold (unmeasured)
import jax
import jax.numpy as jnp
from jax import lax
from jax.experimental import pallas as pl
from jax.experimental.pallas import tpu as pltpu


def kernel(
    x,
):
    def body(*refs):
        pass

    out_shape = jax.ShapeDtypeStruct(..., jnp.float32)
    return pl.pallas_call(body, out_shape=out_shape)(...)


# baseline (device time: 847463 ns/iter reference)
import jax
import jax.numpy as jnp
from jax import lax
from jax.experimental import pallas as pl
from jax.experimental.pallas import tpu as pltpu

M = 16384
N_HALF = 1024
CHUNK = 2048
N_CHUNKS = M // CHUNK


def kernel(x):
    def body(x_hbm, out_hbm, recv_hbm, a_vmem, b_vmem,
             send_sem, recv_sem, load_a_sem, load_b_sem, store_sem):
        mx = lax.axis_index("x")
        my = lax.axis_index("y")
        mz = lax.axis_index("z")
        partner = (1 - mx, my, mz)

        barrier = pltpu.get_barrier_semaphore()
        pl.semaphore_signal(
            barrier, inc=1,
            device_id=partner, device_id_type=pl.DeviceIdType.MESH,
        )
        pl.semaphore_wait(barrier, 1)

        rdma = pltpu.make_async_remote_copy(
            src_ref=x_hbm.at[0, :, pl.ds((1 - mx) * N_HALF, N_HALF)],
            dst_ref=recv_hbm,
            send_sem=send_sem,
            recv_sem=recv_sem,
            device_id=partner,
            device_id_type=pl.DeviceIdType.MESH,
        )
        rdma.start()
        rdma.wait()

        for c in range(N_CHUNKS):
            rows = pl.ds(c * CHUNK, CHUNK)
            cp_a = pltpu.make_async_copy(
                x_hbm.at[0, rows, pl.ds(mx * N_HALF, N_HALF)],
                a_vmem, load_a_sem)
            cp_b = pltpu.make_async_copy(
                recv_hbm.at[rows, :], b_vmem, load_b_sem)
            cp_a.start()
            cp_b.start()
            cp_a.wait()
            cp_b.wait()
            a_vmem[...] = a_vmem[...] + b_vmem[...]
            st = pltpu.make_async_copy(a_vmem, out_hbm.at[rows, :], store_sem)
            st.start()
            st.wait()

    out, _recv = pl.pallas_call(
        body,
        out_shape=[
            jax.ShapeDtypeStruct((M, N_HALF), jnp.float32),
            jax.ShapeDtypeStruct((M, N_HALF), jnp.float32),
        ],
        in_specs=[pl.BlockSpec(memory_space=pl.ANY)],
        out_specs=[
            pl.BlockSpec(memory_space=pl.ANY),
            pl.BlockSpec(memory_space=pl.ANY),
        ],
        scratch_shapes=[
            pltpu.VMEM((CHUNK, N_HALF), jnp.float32),
            pltpu.VMEM((CHUNK, N_HALF), jnp.float32),
            pltpu.SemaphoreType.DMA,
            pltpu.SemaphoreType.DMA,
            pltpu.SemaphoreType.DMA,
            pltpu.SemaphoreType.DMA,
            pltpu.SemaphoreType.DMA,
        ],
        compiler_params=pltpu.CompilerParams(collective_id=0),
    )(x)
    return out


# device time: 437373 ns/iter; 1.9376x vs baseline; 1.9376x over previous
import jax
import jax.numpy as jnp
from jax import lax
from jax.experimental import pallas as pl
from jax.experimental.pallas import tpu as pltpu

M = 16384
N_HALF = 1024
H = M // 2
NCH = 16
CH = H // NCH


def kernel(x):
    def body(x_hbm, out_hbm, xrecv_hbm, va, vb,
             xsend_sems, xrecv_sems, zsend_sems, zrecv_sems,
             la_sems, lb_sems, st_sems):
        mx = lax.axis_index("x")
        my = lax.axis_index("y")
        mz = lax.axis_index("z")
        xp = (1 - mx, my, mz)
        zp = (mx, my, 1 - mz)
        row0 = mz * H

        barrier = pltpu.get_barrier_semaphore()
        for nbr in (xp, zp):
            pl.semaphore_signal(
                barrier, inc=1,
                device_id=nbr, device_id_type=pl.DeviceIdType.MESH,
            )
        pl.semaphore_wait(barrier, 2)

        xrdmas = []
        for c in range(NCH):
            r = row0 + c * CH
            d = pltpu.make_async_remote_copy(
                src_ref=x_hbm.at[0, pl.ds(r, CH),
                                 pl.ds((1 - mx) * N_HALF, N_HALF)],
                dst_ref=xrecv_hbm.at[pl.ds(c * CH, CH), :],
                send_sem=xsend_sems.at[c],
                recv_sem=xrecv_sems.at[c],
                device_id=xp,
                device_id_type=pl.DeviceIdType.MESH,
            )
            d.start()
            xrdmas.append(d)

        strs = [None, None]
        zrds = [None] * NCH
        for c in range(NCH):
            slot = c % 2
            r = row0 + c * CH
            if c >= 2:
                strs[slot].wait()
                zrds[c - 2].wait_send()
            xrdmas[c].wait_recv()
            cp_a = pltpu.make_async_copy(
                x_hbm.at[0, pl.ds(r, CH), pl.ds(mx * N_HALF, N_HALF)],
                va.at[slot], la_sems.at[slot])
            cp_b = pltpu.make_async_copy(
                xrecv_hbm.at[pl.ds(c * CH, CH), :],
                vb.at[slot], lb_sems.at[slot])
            cp_a.start()
            cp_b.start()
            cp_a.wait()
            cp_b.wait()
            va[slot] = va[slot] + vb[slot]
            st = pltpu.make_async_copy(
                va.at[slot], out_hbm.at[pl.ds(r, CH), :], st_sems.at[slot])
            st.start()
            strs[slot] = st
            zd = pltpu.make_async_remote_copy(
                src_ref=va.at[slot],
                dst_ref=out_hbm.at[pl.ds(r, CH), :],
                send_sem=zsend_sems.at[c],
                recv_sem=zrecv_sems.at[c],
                device_id=zp,
                device_id_type=pl.DeviceIdType.MESH,
            )
            zd.start()
            zrds[c] = zd

        strs[(NCH - 2) % 2].wait()
        strs[(NCH - 1) % 2].wait()
        zrds[NCH - 2].wait_send()
        zrds[NCH - 1].wait_send()
        for c in range(NCH):
            zrds[c].wait_recv()
        for c in range(NCH):
            xrdmas[c].wait_send()

    out, _xrecv = pl.pallas_call(
        body,
        out_shape=[
            jax.ShapeDtypeStruct((M, N_HALF), jnp.float32),
            jax.ShapeDtypeStruct((H, N_HALF), jnp.float32),
        ],
        in_specs=[pl.BlockSpec(memory_space=pl.ANY)],
        out_specs=[
            pl.BlockSpec(memory_space=pl.ANY),
            pl.BlockSpec(memory_space=pl.ANY),
        ],
        scratch_shapes=[
            pltpu.VMEM((2, CH, N_HALF), jnp.float32),
            pltpu.VMEM((2, CH, N_HALF), jnp.float32),
            pltpu.SemaphoreType.DMA((NCH,)),
            pltpu.SemaphoreType.DMA((NCH,)),
            pltpu.SemaphoreType.DMA((NCH,)),
            pltpu.SemaphoreType.DMA((NCH,)),
            pltpu.SemaphoreType.DMA((2,)),
            pltpu.SemaphoreType.DMA((2,)),
            pltpu.SemaphoreType.DMA((2,)),
        ],
        compiler_params=pltpu.CompilerParams(collective_id=0),
    )(x)
    return out


# device time: 434021 ns/iter; 1.9526x vs baseline; 1.0077x over previous
import jax
import jax.numpy as jnp
from jax import lax
from jax.experimental import pallas as pl
from jax.experimental.pallas import tpu as pltpu

M = 16384
N_HALF = 1024
H = M // 2
NCH = 16
CH = H // NCH


def kernel(x):
    def body(x_hbm, out_hbm, xrecv_vmem, va,
             xsend_sems, xrecv_sems, zsend_sems, zrecv_sems,
             la_sems, st_sems):
        mx = lax.axis_index("x")
        my = lax.axis_index("y")
        mz = lax.axis_index("z")
        xp = (1 - mx, my, mz)
        zp = (mx, my, 1 - mz)
        row0 = mz * H

        barrier = pltpu.get_barrier_semaphore()
        for nbr in (xp, zp):
            pl.semaphore_signal(
                barrier, inc=1,
                device_id=nbr, device_id_type=pl.DeviceIdType.MESH,
            )
        pl.semaphore_wait(barrier, 2)

        xrdmas = []
        for c in range(NCH):
            r = row0 + c * CH
            d = pltpu.make_async_remote_copy(
                src_ref=x_hbm.at[0, pl.ds(r, CH),
                                 pl.ds((1 - mx) * N_HALF, N_HALF)],
                dst_ref=xrecv_vmem.at[pl.ds(c * CH, CH), :],
                send_sem=xsend_sems.at[c],
                recv_sem=xrecv_sems.at[c],
                device_id=xp,
                device_id_type=pl.DeviceIdType.MESH,
            )
            d.start()
            xrdmas.append(d)

        strs = [None, None]
        zrds = [None] * NCH
        for c in range(NCH):
            slot = c % 2
            r = row0 + c * CH
            if c >= 2:
                strs[slot].wait()
                zrds[c - 2].wait_send()
            cp_a = pltpu.make_async_copy(
                x_hbm.at[0, pl.ds(r, CH), pl.ds(mx * N_HALF, N_HALF)],
                va.at[slot], la_sems.at[slot])
            cp_a.start()
            xrdmas[c].wait_recv()
            cp_a.wait()
            va[slot] = va[slot] + xrecv_vmem[pl.ds(c * CH, CH), :]
            st = pltpu.make_async_copy(
                va.at[slot], out_hbm.at[pl.ds(r, CH), :], st_sems.at[slot])
            st.start()
            strs[slot] = st
            zd = pltpu.make_async_remote_copy(
                src_ref=va.at[slot],
                dst_ref=out_hbm.at[pl.ds(r, CH), :],
                send_sem=zsend_sems.at[c],
                recv_sem=zrecv_sems.at[c],
                device_id=zp,
                device_id_type=pl.DeviceIdType.MESH,
            )
            zd.start()
            zrds[c] = zd

        strs[(NCH - 2) % 2].wait()
        strs[(NCH - 1) % 2].wait()
        zrds[NCH - 2].wait_send()
        zrds[NCH - 1].wait_send()
        for c in range(NCH):
            zrds[c].wait_recv()
        for c in range(NCH):
            xrdmas[c].wait_send()

    out = pl.pallas_call(
        body,
        out_shape=jax.ShapeDtypeStruct((M, N_HALF), jnp.float32),
        in_specs=[pl.BlockSpec(memory_space=pl.ANY)],
        out_specs=pl.BlockSpec(memory_space=pl.ANY),
        scratch_shapes=[
            pltpu.VMEM((H, N_HALF), jnp.float32),
            pltpu.VMEM((2, CH, N_HALF), jnp.float32),
            pltpu.SemaphoreType.DMA((NCH,)),
            pltpu.SemaphoreType.DMA((NCH,)),
            pltpu.SemaphoreType.DMA((NCH,)),
            pltpu.SemaphoreType.DMA((NCH,)),
            pltpu.SemaphoreType.DMA((2,)),
            pltpu.SemaphoreType.DMA((2,)),
        ],
        compiler_params=pltpu.CompilerParams(
            collective_id=0, vmem_limit_bytes=48 * 1024 * 1024),
    )(x)
    return out


# device time: 424146 ns/iter; 1.9980x vs baseline; 1.0233x over previous
import jax
import jax.numpy as jnp
from jax import lax
from jax.experimental import pallas as pl
from jax.experimental.pallas import tpu as pltpu

M = 16384
N_HALF = 1024
H = M // 2
NCH = 32
CH = H // NCH


def kernel(x):
    def body(x_hbm, out_hbm, xrecv_vmem, va,
             xsend_sems, xrecv_sems, zsend_sems, zrecv_sems,
             la_sems, st_sems):
        mx = lax.axis_index("x")
        my = lax.axis_index("y")
        mz = lax.axis_index("z")
        xp = (1 - mx, my, mz)
        zp = (mx, my, 1 - mz)
        row0 = mz * H

        barrier = pltpu.get_barrier_semaphore()
        for nbr in (xp, zp):
            pl.semaphore_signal(
                barrier, inc=1,
                device_id=nbr, device_id_type=pl.DeviceIdType.MESH,
            )
        pl.semaphore_wait(barrier, 2)

        xrdmas = []
        for c in range(NCH):
            r = row0 + c * CH
            d = pltpu.make_async_remote_copy(
                src_ref=x_hbm.at[0, pl.ds(r, CH),
                                 pl.ds((1 - mx) * N_HALF, N_HALF)],
                dst_ref=xrecv_vmem.at[pl.ds(c * CH, CH), :],
                send_sem=xsend_sems.at[c],
                recv_sem=xrecv_sems.at[c],
                device_id=xp,
                device_id_type=pl.DeviceIdType.MESH,
            )
            d.start()
            xrdmas.append(d)

        strs = [None, None]
        zrds = [None] * NCH
        for c in range(NCH):
            slot = c % 2
            r = row0 + c * CH
            if c >= 2:
                strs[slot].wait()
                zrds[c - 2].wait_send()
            cp_a = pltpu.make_async_copy(
                x_hbm.at[0, pl.ds(r, CH), pl.ds(mx * N_HALF, N_HALF)],
                va.at[slot], la_sems.at[slot])
            cp_a.start()
            xrdmas[c].wait_recv()
            cp_a.wait()
            va[slot] = va[slot] + xrecv_vmem[pl.ds(c * CH, CH), :]
            st = pltpu.make_async_copy(
                va.at[slot], out_hbm.at[pl.ds(r, CH), :], st_sems.at[slot])
            st.start()
            strs[slot] = st
            zd = pltpu.make_async_remote_copy(
                src_ref=va.at[slot],
                dst_ref=out_hbm.at[pl.ds(r, CH), :],
                send_sem=zsend_sems.at[c],
                recv_sem=zrecv_sems.at[c],
                device_id=zp,
                device_id_type=pl.DeviceIdType.MESH,
            )
            zd.start()
            zrds[c] = zd

        strs[(NCH - 2) % 2].wait()
        strs[(NCH - 1) % 2].wait()
        zrds[NCH - 2].wait_send()
        zrds[NCH - 1].wait_send()
        for c in range(NCH):
            zrds[c].wait_recv()
        for c in range(NCH):
            xrdmas[c].wait_send()

    out = pl.pallas_call(
        body,
        out_shape=jax.ShapeDtypeStruct((M, N_HALF), jnp.float32),
        in_specs=[pl.BlockSpec(memory_space=pl.ANY)],
        out_specs=pl.BlockSpec(memory_space=pl.ANY),
        scratch_shapes=[
            pltpu.VMEM((H, N_HALF), jnp.float32),
            pltpu.VMEM((2, CH, N_HALF), jnp.float32),
            pltpu.SemaphoreType.DMA((NCH,)),
            pltpu.SemaphoreType.DMA((NCH,)),
            pltpu.SemaphoreType.DMA((NCH,)),
            pltpu.SemaphoreType.DMA((NCH,)),
            pltpu.SemaphoreType.DMA((2,)),
            pltpu.SemaphoreType.DMA((2,)),
        ],
        compiler_params=pltpu.CompilerParams(
            collective_id=0, vmem_limit_bytes=48 * 1024 * 1024),
    )(x)
    return out


# device time: 393305 ns/iter; 2.1547x vs baseline; 1.0784x over previous
import jax
import jax.numpy as jnp
from jax import lax
from jax.experimental import pallas as pl
from jax.experimental.pallas import tpu as pltpu

M = 16384
N_HALF = 1024
Q = M // 4
NC = 16
CH = Q // NC
NF = NC // 2
LAG = 2


def kernel(x):
    def body(x_hbm, out_hbm, xrecv_vmem, va,
             xsend_sems, xrecv_sems, ysend_sems, yrecv_sems,
             zsend_sems, zrecv_sems, la_sems, st_sems):
        mx = lax.axis_index("x")
        my = lax.axis_index("y")
        mz = lax.axis_index("z")
        xp = (1 - mx, my, mz)
        yp = (mx, 1 - my, mz)
        zp = (mx, my, 1 - mz)
        row0 = (my * 2 + mz) * Q
        rowy0 = ((1 - my) * 2 + mz) * Q
        rowz0 = (my * 2 + (1 - mz)) * Q

        barrier = pltpu.get_barrier_semaphore()
        for nbr in (xp, yp, zp):
            pl.semaphore_signal(
                barrier, inc=1,
                device_id=nbr, device_id_type=pl.DeviceIdType.MESH,
            )
        pl.semaphore_wait(barrier, 3)

        xr = []
        for c in range(NC):
            r = row0 + c * CH
            d = pltpu.make_async_remote_copy(
                src_ref=x_hbm.at[0, pl.ds(r, CH),
                                 pl.ds((1 - mx) * N_HALF, N_HALF)],
                dst_ref=xrecv_vmem.at[pl.ds(c * CH, CH), :],
                send_sem=xsend_sems.at[c],
                recv_sem=xrecv_sems.at[c],
                device_id=xp,
                device_id_type=pl.DeviceIdType.MESH,
            )
            d.start()
            xr.append(d)

        sts = [None, None]
        yd = [None] * NC
        zd = [None] * NC
        fy = [None] * NF
        fz = [None] * NF

        def fwd_step(j):
            if 0 <= j < NF:
                yd_rows = pl.ds(rowy0 + j * CH, CH)
                yd[j].wait_recv()
                f = pltpu.make_async_remote_copy(
                    src_ref=out_hbm.at[yd_rows, :],
                    dst_ref=out_hbm.at[yd_rows, :],
                    send_sem=zsend_sems.at[NC + j],
                    recv_sem=zrecv_sems.at[NC + j],
                    device_id=zp,
                    device_id_type=pl.DeviceIdType.MESH,
                )
                f.start()
                fz[j] = f
            elif NF <= j < NC:
                zd_rows = pl.ds(rowz0 + j * CH, CH)
                zd[j].wait_recv()
                f = pltpu.make_async_remote_copy(
                    src_ref=out_hbm.at[zd_rows, :],
                    dst_ref=out_hbm.at[zd_rows, :],
                    send_sem=ysend_sems.at[NC + (j - NF)],
                    recv_sem=yrecv_sems.at[NC + (j - NF)],
                    device_id=yp,
                    device_id_type=pl.DeviceIdType.MESH,
                )
                f.start()
                fy[j - NF] = f

        for c in range(NC):
            slot = c % 2
            r = row0 + c * CH
            rows = pl.ds(r, CH)
            if c >= 2:
                sts[slot].wait()
                yd[c - 2].wait_send()
                zd[c - 2].wait_send()
            cp_a = pltpu.make_async_copy(
                x_hbm.at[0, rows, pl.ds(mx * N_HALF, N_HALF)],
                va.at[slot], la_sems.at[slot])
            cp_a.start()
            xr[c].wait_recv()
            cp_a.wait()
            va[slot] = va[slot] + xrecv_vmem[pl.ds(c * CH, CH), :]
            st = pltpu.make_async_copy(
                va.at[slot], out_hbm.at[rows, :], st_sems.at[slot])
            st.start()
            sts[slot] = st
            yd[c] = pltpu.make_async_remote_copy(
                src_ref=va.at[slot], dst_ref=out_hbm.at[rows, :],
                send_sem=ysend_sems.at[c], recv_sem=yrecv_sems.at[c],
                device_id=yp, device_id_type=pl.DeviceIdType.MESH,
            )
            yd[c].start()
            zd[c] = pltpu.make_async_remote_copy(
                src_ref=va.at[slot], dst_ref=out_hbm.at[rows, :],
                send_sem=zsend_sems.at[c], recv_sem=zrecv_sems.at[c],
                device_id=zp, device_id_type=pl.DeviceIdType.MESH,
            )
            zd[c].start()
            fwd_step(c - LAG)

        for c in range(NC, NC + LAG):
            fwd_step(c - LAG)

        sts[0].wait()
        sts[1].wait()
        for c in range(NC - 2, NC):
            yd[c].wait_send()
            zd[c].wait_send()
        for j in range(NF):
            fz[j].wait_send()
            fy[j].wait_send()
        for j in range(NF, NC):
            yd[j].wait_recv()
        for j in range(NF):
            zd[j].wait_recv()
        for j in range(NF):
            fz[j].wait_recv()
            fy[j].wait_recv()
        for c in range(NC):
            xr[c].wait_send()

    out = pl.pallas_call(
        body,
        out_shape=jax.ShapeDtypeStruct((M, N_HALF), jnp.float32),
        in_specs=[pl.BlockSpec(memory_space=pl.ANY)],
        out_specs=pl.BlockSpec(memory_space=pl.ANY),
        scratch_shapes=[
            pltpu.VMEM((Q, N_HALF), jnp.float32),
            pltpu.VMEM((2, CH, N_HALF), jnp.float32),
            pltpu.SemaphoreType.DMA((NC,)),
            pltpu.SemaphoreType.DMA((NC,)),
            pltpu.SemaphoreType.DMA((NC + NF,)),
            pltpu.SemaphoreType.DMA((NC + NF,)),
            pltpu.SemaphoreType.DMA((NC + NF,)),
            pltpu.SemaphoreType.DMA((NC + NF,)),
            pltpu.SemaphoreType.DMA((2,)),
            pltpu.SemaphoreType.DMA((2,)),
        ],
        compiler_params=pltpu.CompilerParams(
            collective_id=0, vmem_limit_bytes=48 * 1024 * 1024),
    )(x)
    return out


# device time: 336816 ns/iter; 2.5161x vs baseline; 1.1677x over previous
import jax
import jax.numpy as jnp
from jax import lax
from jax.experimental import pallas as pl
from jax.experimental.pallas import tpu as pltpu

M = 16384
N_HALF = 1024
Q = M // 4
NC = 16
CH = Q // NC
NF = NC // 2
LAG = 2
NSLOT = 4


def kernel(x):
    def body(x_hbm, out_hbm, xrecv_vmem, va,
             xsend_sems, xrecv_sems, ysend_sems, yrecv_sems,
             zsend_sems, zrecv_sems, la_sems, st_sems):
        mx = lax.axis_index("x")
        my = lax.axis_index("y")
        mz = lax.axis_index("z")
        xp = (1 - mx, my, mz)
        yp = (mx, 1 - my, mz)
        zp = (mx, my, 1 - mz)
        row0 = (my * 2 + mz) * Q
        rowy0 = ((1 - my) * 2 + mz) * Q
        rowz0 = (my * 2 + (1 - mz)) * Q

        barrier = pltpu.get_barrier_semaphore()
        for nbr in (xp, yp, zp):
            pl.semaphore_signal(
                barrier, inc=1,
                device_id=nbr, device_id_type=pl.DeviceIdType.MESH,
            )
        pl.semaphore_wait(barrier, 3)

        xr = []
        for c in range(NC):
            r = row0 + c * CH
            d = pltpu.make_async_remote_copy(
                src_ref=x_hbm.at[0, pl.ds(r, CH),
                                 pl.ds((1 - mx) * N_HALF, N_HALF)],
                dst_ref=xrecv_vmem.at[pl.ds(c * CH, CH), :],
                send_sem=xsend_sems.at[c],
                recv_sem=xrecv_sems.at[c],
                device_id=xp,
                device_id_type=pl.DeviceIdType.MESH,
            )
            d.start()
            xr.append(d)

        sts = [None] * NSLOT
        yd = [None] * NC
        zd = [None] * NC
        fy = [None] * NF
        fz = [None] * NF

        def fwd_step(j):
            if not (0 <= j < NC):
                return
            if j % 2 == 0:
                rows = pl.ds(rowy0 + j * CH, CH)
                yd[j].wait_recv()
                f = pltpu.make_async_remote_copy(
                    src_ref=out_hbm.at[rows, :],
                    dst_ref=out_hbm.at[rows, :],
                    send_sem=zsend_sems.at[NC + j // 2],
                    recv_sem=zrecv_sems.at[NC + j // 2],
                    device_id=zp,
                    device_id_type=pl.DeviceIdType.MESH,
                )
                f.start()
                fz[j // 2] = f
            else:
                rows = pl.ds(rowz0 + j * CH, CH)
                zd[j].wait_recv()
                f = pltpu.make_async_remote_copy(
                    src_ref=out_hbm.at[rows, :],
                    dst_ref=out_hbm.at[rows, :],
                    send_sem=ysend_sems.at[NC + j // 2],
                    recv_sem=yrecv_sems.at[NC + j // 2],
                    device_id=yp,
                    device_id_type=pl.DeviceIdType.MESH,
                )
                f.start()
                fy[j // 2] = f

        for c in range(NC):
            slot = c % NSLOT
            r = row0 + c * CH
            rows = pl.ds(r, CH)
            if c >= NSLOT:
                sts[slot].wait()
                yd[c - NSLOT].wait_send()
                zd[c - NSLOT].wait_send()
            cp_a = pltpu.make_async_copy(
                x_hbm.at[0, rows, pl.ds(mx * N_HALF, N_HALF)],
                va.at[slot], la_sems.at[slot])
            cp_a.start()
            xr[c].wait_recv()
            cp_a.wait()
            va[slot] = va[slot] + xrecv_vmem[pl.ds(c * CH, CH), :]
            st = pltpu.make_async_copy(
                va.at[slot], out_hbm.at[rows, :], st_sems.at[slot])
            st.start()
            sts[slot] = st
            yd[c] = pltpu.make_async_remote_copy(
                src_ref=va.at[slot], dst_ref=out_hbm.at[rows, :],
                send_sem=ysend_sems.at[c], recv_sem=yrecv_sems.at[c],
                device_id=yp, device_id_type=pl.DeviceIdType.MESH,
            )
            yd[c].start()
            zd[c] = pltpu.make_async_remote_copy(
                src_ref=va.at[slot], dst_ref=out_hbm.at[rows, :],
                send_sem=zsend_sems.at[c], recv_sem=zrecv_sems.at[c],
                device_id=zp, device_id_type=pl.DeviceIdType.MESH,
            )
            zd[c].start()
            fwd_step(c - LAG)

        for c in range(NC, NC + LAG):
            fwd_step(c - LAG)

        for s in sts:
            s.wait()
        for c in range(NC - NSLOT, NC):
            yd[c].wait_send()
            zd[c].wait_send()
        for j in range(NF):
            fz[j].wait_send()
            fy[j].wait_send()
        for j in range(1, NC, 2):
            yd[j].wait_recv()
        for j in range(0, NC, 2):
            zd[j].wait_recv()
        for j in range(NF):
            fz[j].wait_recv()
            fy[j].wait_recv()
        for c in range(NC):
            xr[c].wait_send()

    out = pl.pallas_call(
        body,
        out_shape=jax.ShapeDtypeStruct((M, N_HALF), jnp.float32),
        in_specs=[pl.BlockSpec(memory_space=pl.ANY)],
        out_specs=pl.BlockSpec(memory_space=pl.ANY),
        scratch_shapes=[
            pltpu.VMEM((Q, N_HALF), jnp.float32),
            pltpu.VMEM((NSLOT, CH, N_HALF), jnp.float32),
            pltpu.SemaphoreType.DMA((NC,)),
            pltpu.SemaphoreType.DMA((NC,)),
            pltpu.SemaphoreType.DMA((NC + NF,)),
            pltpu.SemaphoreType.DMA((NC + NF,)),
            pltpu.SemaphoreType.DMA((NC + NF,)),
            pltpu.SemaphoreType.DMA((NC + NF,)),
            pltpu.SemaphoreType.DMA((NSLOT,)),
            pltpu.SemaphoreType.DMA((NSLOT,)),
        ],
        compiler_params=pltpu.CompilerParams(
            collective_id=0, vmem_limit_bytes=48 * 1024 * 1024),
    )(x)
    return out


# device time: 311551 ns/iter; 2.7201x vs baseline; 1.0811x over previous
import jax
import jax.numpy as jnp
from jax import lax
from jax.experimental import pallas as pl
from jax.experimental.pallas import tpu as pltpu

M = 16384
N_HALF = 1024
Q = M // 4
NC = 16
CH = Q // NC
LAG = 2
NSLOT = 4

FZ_LIST = [j for j in range(NC) if j % 3 == 0]
FY_LIST = [j for j in range(NC) if j % 3 == 1]
DS_LIST = [j for j in range(NC) if j % 3 == 2]
NFZ = len(FZ_LIST)
NFY = len(FY_LIST)
NDS = len(DS_LIST)
NT = NC + NDS


def kernel(x):
    def body(x_hbm, out_hbm, xrecv_vmem, va,
             xsend_sems, xrecv_sems, ysend_sems, yrecv_sems,
             zsend_sems, zrecv_sems, la_sems, st_sems):
        mx = lax.axis_index("x")
        my = lax.axis_index("y")
        mz = lax.axis_index("z")
        xp = (1 - mx, my, mz)
        yp = (mx, 1 - my, mz)
        zp = (mx, my, 1 - mz)
        row0 = (my * 2 + mz) * Q
        rowy0 = ((1 - my) * 2 + mz) * Q
        rowz0 = (my * 2 + (1 - mz)) * Q
        rowd0 = ((1 - my) * 2 + (1 - mz)) * Q

        barrier = pltpu.get_barrier_semaphore()
        for nbr in (xp, yp, zp):
            pl.semaphore_signal(
                barrier, inc=1,
                device_id=nbr, device_id_type=pl.DeviceIdType.MESH,
            )
        pl.semaphore_wait(barrier, 3)

        xr = []
        for i in range(NT):
            r = (row0 + i * CH) if i < NC else (rowd0 + DS_LIST[i - NC] * CH)
            d = pltpu.make_async_remote_copy(
                src_ref=x_hbm.at[0, pl.ds(r, CH),
                                 pl.ds((1 - mx) * N_HALF, N_HALF)],
                dst_ref=xrecv_vmem.at[pl.ds(i * CH, CH), :],
                send_sem=xsend_sems.at[i],
                recv_sem=xrecv_sems.at[i],
                device_id=xp,
                device_id_type=pl.DeviceIdType.MESH,
            )
            d.start()
            xr.append(d)

        sts = [None] * NSLOT
        yd = [None] * NC
        zd = [None] * NC
        fy = [None] * NFY
        fz = [None] * NFZ

        def fwd_step(j):
            if not (0 <= j < NC):
                return
            if j % 3 == 0:
                rows = pl.ds(rowy0 + j * CH, CH)
                yd[j].wait_recv()
                k = j // 3
                f = pltpu.make_async_remote_copy(
                    src_ref=out_hbm.at[rows, :],
                    dst_ref=out_hbm.at[rows, :],
                    send_sem=zsend_sems.at[NC + k],
                    recv_sem=zrecv_sems.at[NC + k],
                    device_id=zp,
                    device_id_type=pl.DeviceIdType.MESH,
                )
                f.start()
                fz[k] = f
            elif j % 3 == 1:
                rows = pl.ds(rowz0 + j * CH, CH)
                zd[j].wait_recv()
                k = j // 3
                f = pltpu.make_async_remote_copy(
                    src_ref=out_hbm.at[rows, :],
                    dst_ref=out_hbm.at[rows, :],
                    send_sem=ysend_sems.at[NC + k],
                    recv_sem=yrecv_sems.at[NC + k],
                    device_id=yp,
                    device_id_type=pl.DeviceIdType.MESH,
                )
                f.start()
                fy[k] = f

        for c in range(NT):
            slot = c % NSLOT
            r = (row0 + c * CH) if c < NC else (rowd0 + DS_LIST[c - NC] * CH)
            rows = pl.ds(r, CH)
            if c >= NSLOT:
                sts[slot].wait()
                if c - NSLOT < NC:
                    yd[c - NSLOT].wait_send()
                    zd[c - NSLOT].wait_send()
            cp_a = pltpu.make_async_copy(
                x_hbm.at[0, rows, pl.ds(mx * N_HALF, N_HALF)],
                va.at[slot], la_sems.at[slot])
            cp_a.start()
            xr[c].wait_recv()
            cp_a.wait()
            va[slot] = va[slot] + xrecv_vmem[pl.ds(c * CH, CH), :]
            st = pltpu.make_async_copy(
                va.at[slot], out_hbm.at[rows, :], st_sems.at[slot])
            st.start()
            sts[slot] = st
            if c < NC:
                yd[c] = pltpu.make_async_remote_copy(
                    src_ref=va.at[slot], dst_ref=out_hbm.at[rows, :],
                    send_sem=ysend_sems.at[c], recv_sem=yrecv_sems.at[c],
                    device_id=yp, device_id_type=pl.DeviceIdType.MESH,
                )
                yd[c].start()
                zd[c] = pltpu.make_async_remote_copy(
                    src_ref=va.at[slot], dst_ref=out_hbm.at[rows, :],
                    send_sem=zsend_sems.at[c], recv_sem=zrecv_sems.at[c],
                    device_id=zp, device_id_type=pl.DeviceIdType.MESH,
                )
                zd[c].start()
            fwd_step(c - LAG)

        for c in range(NT, NC + LAG):
            fwd_step(c - LAG)

        for s in sts:
            s.wait()
        for k in range(NFZ):
            fz[k].wait_send()
            fz[k].wait_recv()
        for k in range(NFY):
            fy[k].wait_send()
            fy[k].wait_recv()
        for j in range(NC):
            if j % 3 != 0:
                yd[j].wait_recv()
            if j % 3 != 1:
                zd[j].wait_recv()
        for i in range(NT):
            xr[i].wait_send()

    out = pl.pallas_call(
        body,
        out_shape=jax.ShapeDtypeStruct((M, N_HALF), jnp.float32),
        in_specs=[pl.BlockSpec(memory_space=pl.ANY)],
        out_specs=pl.BlockSpec(memory_space=pl.ANY),
        scratch_shapes=[
            pltpu.VMEM((NT * CH, N_HALF), jnp.float32),
            pltpu.VMEM((NSLOT, CH, N_HALF), jnp.float32),
            pltpu.SemaphoreType.DMA((NT,)),
            pltpu.SemaphoreType.DMA((NT,)),
            pltpu.SemaphoreType.DMA((NC + NFY,)),
            pltpu.SemaphoreType.DMA((NC + NFY,)),
            pltpu.SemaphoreType.DMA((NC + NFZ,)),
            pltpu.SemaphoreType.DMA((NC + NFZ,)),
            pltpu.SemaphoreType.DMA((NSLOT,)),
            pltpu.SemaphoreType.DMA((NSLOT,)),
        ],
        compiler_params=pltpu.CompilerParams(
            collective_id=0, vmem_limit_bytes=48 * 1024 * 1024),
    )(x)
    return out


# device time: 307971 ns/iter; 2.7518x vs baseline; 1.0116x over previous
import jax
import jax.numpy as jnp
from jax import lax
from jax.experimental import pallas as pl
from jax.experimental.pallas import tpu as pltpu

M = 16384
N_HALF = 1024
Q = M // 4
NC = 32
CH = Q // NC
LAG = 2
NSLOT = 4

FZ_LIST = [j for j in range(NC) if j % 3 == 0]
FY_LIST = [j for j in range(NC) if j % 3 == 1]
DS_LIST = [j for j in range(NC) if j % 3 == 2]
NFZ = len(FZ_LIST)
NFY = len(FY_LIST)
NDS = len(DS_LIST)
NT = NC + NDS


def kernel(x):
    def body(x_hbm, out_hbm, xrecv_vmem, va,
             xsend_sems, xrecv_sems, ysend_sems, yrecv_sems,
             zsend_sems, zrecv_sems, la_sems, st_sems):
        mx = lax.axis_index("x")
        my = lax.axis_index("y")
        mz = lax.axis_index("z")
        xp = (1 - mx, my, mz)
        yp = (mx, 1 - my, mz)
        zp = (mx, my, 1 - mz)
        row0 = (my * 2 + mz) * Q
        rowy0 = ((1 - my) * 2 + mz) * Q
        rowz0 = (my * 2 + (1 - mz)) * Q
        rowd0 = ((1 - my) * 2 + (1 - mz)) * Q

        barrier = pltpu.get_barrier_semaphore()
        for nbr in (xp, yp, zp):
            pl.semaphore_signal(
                barrier, inc=1,
                device_id=nbr, device_id_type=pl.DeviceIdType.MESH,
            )
        pl.semaphore_wait(barrier, 3)

        xr = []
        for i in range(NT):
            r = (row0 + i * CH) if i < NC else (rowd0 + DS_LIST[i - NC] * CH)
            d = pltpu.make_async_remote_copy(
                src_ref=x_hbm.at[0, pl.ds(r, CH),
                                 pl.ds((1 - mx) * N_HALF, N_HALF)],
                dst_ref=xrecv_vmem.at[pl.ds(i * CH, CH), :],
                send_sem=xsend_sems.at[i],
                recv_sem=xrecv_sems.at[i],
                device_id=xp,
                device_id_type=pl.DeviceIdType.MESH,
            )
            d.start()
            xr.append(d)

        sts = [None] * NSLOT
        yd = [None] * NC
        zd = [None] * NC
        fy = [None] * NFY
        fz = [None] * NFZ

        def fwd_step(j):
            if not (0 <= j < NC):
                return
            if j % 3 == 0:
                rows = pl.ds(rowy0 + j * CH, CH)
                yd[j].wait_recv()
                k = j // 3
                f = pltpu.make_async_remote_copy(
                    src_ref=out_hbm.at[rows, :],
                    dst_ref=out_hbm.at[rows, :],
                    send_sem=zsend_sems.at[NC + k],
                    recv_sem=zrecv_sems.at[NC + k],
                    device_id=zp,
                    device_id_type=pl.DeviceIdType.MESH,
                )
                f.start()
                fz[k] = f
            elif j % 3 == 1:
                rows = pl.ds(rowz0 + j * CH, CH)
                zd[j].wait_recv()
                k = j // 3
                f = pltpu.make_async_remote_copy(
                    src_ref=out_hbm.at[rows, :],
                    dst_ref=out_hbm.at[rows, :],
                    send_sem=ysend_sems.at[NC + k],
                    recv_sem=yrecv_sems.at[NC + k],
                    device_id=yp,
                    device_id_type=pl.DeviceIdType.MESH,
                )
                f.start()
                fy[k] = f

        for c in range(NT):
            slot = c % NSLOT
            r = (row0 + c * CH) if c < NC else (rowd0 + DS_LIST[c - NC] * CH)
            rows = pl.ds(r, CH)
            if c >= NSLOT:
                sts[slot].wait()
                if c - NSLOT < NC:
                    yd[c - NSLOT].wait_send()
                    zd[c - NSLOT].wait_send()
            cp_a = pltpu.make_async_copy(
                x_hbm.at[0, rows, pl.ds(mx * N_HALF, N_HALF)],
                va.at[slot], la_sems.at[slot])
            cp_a.start()
            xr[c].wait_recv()
            cp_a.wait()
            va[slot] = va[slot] + xrecv_vmem[pl.ds(c * CH, CH), :]
            st = pltpu.make_async_copy(
                va.at[slot], out_hbm.at[rows, :], st_sems.at[slot])
            st.start()
            sts[slot] = st
            if c < NC:
                yd[c] = pltpu.make_async_remote_copy(
                    src_ref=va.at[slot], dst_ref=out_hbm.at[rows, :],
                    send_sem=ysend_sems.at[c], recv_sem=yrecv_sems.at[c],
                    device_id=yp, device_id_type=pl.DeviceIdType.MESH,
                )
                yd[c].start()
                zd[c] = pltpu.make_async_remote_copy(
                    src_ref=va.at[slot], dst_ref=out_hbm.at[rows, :],
                    send_sem=zsend_sems.at[c], recv_sem=zrecv_sems.at[c],
                    device_id=zp, device_id_type=pl.DeviceIdType.MESH,
                )
                zd[c].start()
            fwd_step(c - LAG)

        for c in range(NT, NC + LAG):
            fwd_step(c - LAG)

        for s in sts:
            s.wait()
        for k in range(NFZ):
            fz[k].wait_send()
            fz[k].wait_recv()
        for k in range(NFY):
            fy[k].wait_send()
            fy[k].wait_recv()
        for j in range(NC):
            if j % 3 != 0:
                yd[j].wait_recv()
            if j % 3 != 1:
                zd[j].wait_recv()
        for i in range(NT):
            xr[i].wait_send()

    out = pl.pallas_call(
        body,
        out_shape=jax.ShapeDtypeStruct((M, N_HALF), jnp.float32),
        in_specs=[pl.BlockSpec(memory_space=pl.ANY)],
        out_specs=pl.BlockSpec(memory_space=pl.ANY),
        scratch_shapes=[
            pltpu.VMEM((NT * CH, N_HALF), jnp.float32),
            pltpu.VMEM((NSLOT, CH, N_HALF), jnp.float32),
            pltpu.SemaphoreType.DMA((NT,)),
            pltpu.SemaphoreType.DMA((NT,)),
            pltpu.SemaphoreType.DMA((NC + NFY,)),
            pltpu.SemaphoreType.DMA((NC + NFY,)),
            pltpu.SemaphoreType.DMA((NC + NFZ,)),
            pltpu.SemaphoreType.DMA((NC + NFZ,)),
            pltpu.SemaphoreType.DMA((NSLOT,)),
            pltpu.SemaphoreType.DMA((NSLOT,)),
        ],
        compiler_params=pltpu.CompilerParams(
            collective_id=0, vmem_limit_bytes=48 * 1024 * 1024),
    )(x)
    return out
